# Initial kernel scaffold; baseline (speedup 1.0000x reference)
import jax
import jax.numpy as jnp
from jax.experimental import pallas as pl


def kernel(coords, coord_mask, residue_type):
    raise NotImplementedError("write your pallas kernel here")


# speedup vs baseline: 51.2500x; 51.2500x over previous
import functools

import numpy as np
import jax
import jax.numpy as jnp
from jax import lax
from jax.experimental import pallas as pl
from jax.experimental.pallas import tpu as pltpu
from jax.experimental.pallas import tpu_sc as plsc

_CHAIN_X = np.zeros((21, 4), np.int32)
_NUM_CHI = np.zeros((21,), np.int32)
for _aa, _xs in {
    1: [5, 11, 23, 33],
    2: [5, 16],
    3: [5, 16],
    4: [10],
    5: [5, 11, 26],
    6: [5, 11, 26],
    8: [5, 14],
    9: [6, 12],
    10: [5, 12],
    11: [5, 11, 19, 31],
    12: [5, 18, 19],
    13: [5, 12],
    14: [5, 11],
    15: [8],
    16: [9],
    17: [5, 12],
    18: [5, 12],
    19: [6],
}.items():
    _CHAIN_X[_aa, : len(_xs)] = _xs
    _NUM_CHI[_aa] = len(_xs)

_XTAB = np.zeros((96,), np.int32)
_XTAB[:84] = _CHAIN_X.reshape(-1)
_CTAB = np.zeros((96,), np.float32)
_CTAB[:84] = (np.arange(4)[None, :] < _NUM_CHI[:, None]).astype(np.float32).reshape(-1)

_LIMS = np.linspace(-np.pi, np.pi, 20).astype(np.float32).astype(np.float64)
_sl, _cl = np.sin(_LIMS), np.cos(_LIMS)
_r = _sl / (np.abs(_cl) + np.abs(_sl))
_PL = np.where(_cl >= 0, _r, np.where(_sl >= 0, 2.0 - _r, -2.0 - _r))
_PL_LIST = [float(np.float32(v)) for v in _PL]

_NC, _NS, _L = 2, 16, 16
_NW = _NC * _NS
_BN = 32 * 512
_RPW = _BN // _NW
_C = 256
_F = 88


def _cross(a, b):
    return [a[1] * b[2] - a[2] * b[1],
            a[2] * b[0] - a[0] * b[2],
            a[0] * b[1] - a[1] * b[0]]


def _dot3(a, b):
    return a[0] * b[0] + a[1] * b[1] + a[2] * b[2]


def _sc_body(coords_hbm, cmask_hbm, rt_hbm, xtab_hbm, ctab_hbm, out_hbm,
             coords_v, cmask_v, rt_v, xtab_v, ctab_v, out_v):
    wid = lax.axis_index("s") * _NC + lax.axis_index("c")
    base = wid * _RPW
    pltpu.sync_copy(xtab_hbm, xtab_v)
    pltpu.sync_copy(ctab_hbm, ctab_v)

    lane = lax.iota(jnp.int32, _L)
    zeros = jnp.zeros((_L,), jnp.float32)
    ones = jnp.full((_L,), 1.0, jnp.float32)

    def body(g, carry):
        ridx = g * _L + lane
        rt = plsc.load_gather(rt_v, [ridx])
        rt = lax.min(lax.max(rt, jnp.full((_L,), 0, jnp.int32)),
                     jnp.full((_L,), 20, jnp.int32))
        rt4 = rt * 4
        atoms = [jnp.full((_L,), a, jnp.int32) for a in (0, 1, 3)]
        atoms += [plsc.load_gather(xtab_v, [rt4 + j]) for j in range(4)]
        P = [[plsc.load_gather(coords_v, [ridx, atoms[j] * 3 + c])
              for c in range(3)] for j in range(7)]
        M = [plsc.load_gather(cmask_v, [ridx, atoms[j]]) for j in range(7)]
        CM = [plsc.load_gather(ctab_v, [rt4 + s]) for s in range(4)]

        zbase = g * (_L * _F)
        for k in range(_F):
            out_v[pl.ds(zbase + k * _L, _L)] = zeros

        B = [[P[j + 1][c] - P[j][c] for c in range(3)] for j in range(6)]
        N = [_cross(B[j], B[j + 1]) for j in range(5)]
        rbase = ridx * _F
        for s in range(4):
            n1, n2, b2 = N[s], N[s + 1], B[s + 1]
            x = _dot3(n1, n2)
            yv = _dot3(_cross(n1, b2), n2)
            nu2 = _dot3(b2, b2)
            i = plsc.bitcast(nu2, jnp.int32)
            r = plsc.bitcast(jnp.int32(0x5F3759DF) - (i >> 1), jnp.float32)
            for _ in range(3):
                r = r * (1.5 - 0.5 * nu2 * r * r)
            nu = nu2 * r
            y = yv / (nu + 1e-10)
            pr = y / (jnp.abs(x) + jnp.abs(y))
            p = jnp.where(x >= 0, pr,
                          jnp.where(y >= 0, 2.0 - pr, -2.0 - pr))
            cnt = jnp.zeros((_L,), jnp.int32)
            for t in _PL_LIST:
                cnt = cnt + (p > t).astype(jnp.int32)
            am = M[s] * M[s + 1] * M[s + 2] * M[s + 3]
            on = (CM[s] * am) > 0.5
            plsc.store_scatter(out_v, [rbase + (21 * s) + cnt], ones, mask=on)
            onf = jnp.where(on, 1.0, 0.0).astype(jnp.float32)
            plsc.store_scatter(out_v, [rbase + (84 + s)], onf)
        return carry

    for t in range(_RPW // _C):
        cbase = base + t * _C
        pltpu.sync_copy(coords_hbm.at[pl.ds(cbase, _C)], coords_v)
        pltpu.sync_copy(cmask_hbm.at[pl.ds(cbase, _C)], cmask_v)
        pltpu.sync_copy(rt_hbm.at[pl.ds(cbase, _C)], rt_v)
        lax.fori_loop(0, _C // _L, body, 0)
        pltpu.sync_copy(out_v, out_hbm.at[pl.ds(cbase * _F, _C * _F)])


@jax.jit
def _run(coords_f, cmask_f, rt_f, xtab, ctab):
    mesh = plsc.VectorSubcoreMesh(core_axis_name="c", subcore_axis_name="s")
    return pl.kernel(
        _sc_body,
        out_type=jax.ShapeDtypeStruct((_BN * _F,), jnp.float32),
        mesh=mesh,
        compiler_params=pltpu.CompilerParams(needs_layout_passes=False),
        scratch_types=[
            pltpu.VMEM((_C, 111), jnp.float32),
            pltpu.VMEM((_C, 37), jnp.float32),
            pltpu.VMEM((_C,), jnp.int32),
            pltpu.VMEM((96,), jnp.int32),
            pltpu.VMEM((96,), jnp.float32),
            pltpu.VMEM((_C * _F,), jnp.float32),
        ],
    )(coords_f, cmask_f, rt_f, xtab, ctab)


def kernel(coords, coord_mask, residue_type):
    b, n = residue_type.shape
    coords_f = coords.reshape(b * n, 37 * 3)
    cmask_f = coord_mask.reshape(b * n, 37)
    rt_f = residue_type.reshape(b * n).astype(jnp.int32)
    out = _run(coords_f, cmask_f, rt_f, jnp.asarray(_XTAB), jnp.asarray(_CTAB))
    return out.reshape(b, n, _F)
